# baseline (speedup 1.0000x reference)
import functools

import jax
import jax.numpy as jnp
from jax import lax
from jax.experimental import pallas as pl
from jax.experimental.pallas import tpu as pltpu
from jax.experimental.pallas import tpu_sc as plsc

U_DIM = 64
I_DIM = 64


def _sc_gather(user_table2, uphys, item_table2, iphys):
    B = uphys.shape[0]
    info = plsc.get_sparse_core_info()
    NC, NS = info.num_cores, info.num_subcores
    NW = NC * NS
    b_per_w = B // NW
    mesh = plsc.VectorSubcoreMesh(core_axis_name="c", subcore_axis_name="s")

    @functools.partial(
        pl.kernel,
        mesh=mesh,
        out_type=(
            jax.ShapeDtypeStruct((B, 2 * U_DIM), jnp.float32),
            jax.ShapeDtypeStruct((B, 2 * I_DIM), jnp.float32),
        ),
        scratch_types=[
            pltpu.VMEM((b_per_w,), jnp.int32),
            pltpu.VMEM((b_per_w,), jnp.int32),
            pltpu.VMEM((b_per_w, 2 * U_DIM), jnp.float32),
            pltpu.SemaphoreType.DMA,
        ],
    )
    def gather_k(ut_hbm, uid_hbm, it_hbm, iid_hbm, out_u_hbm, out_i_hbm,
                 uidx_v, iidx_v, rows_v, sem):
        wid = lax.axis_index("s") * NC + lax.axis_index("c")
        base = wid * b_per_w
        pltpu.sync_copy(uid_hbm.at[pl.ds(base, b_per_w)], uidx_v)
        pltpu.sync_copy(iid_hbm.at[pl.ds(base, b_per_w)], iidx_v)
        pltpu.async_copy(ut_hbm.at[uidx_v], rows_v, sem).wait()
        pltpu.sync_copy(rows_v, out_u_hbm.at[pl.ds(base, b_per_w)])
        pltpu.async_copy(it_hbm.at[iidx_v], rows_v, sem).wait()
        pltpu.sync_copy(rows_v, out_i_hbm.at[pl.ds(base, b_per_w)])

    return gather_k(user_table2, uphys, item_table2, iphys)


def _tc_combine(ue2, uhalf, uf, ie2, ihalf, W_uf, b_uf, W_final, b_final):
    B, ufd = uf.shape
    blk = 2048
    n_tasks = W_final.shape[1]

    def body(ue_ref, uh_ref, uf_ref, ie_ref, ih_ref, wuf_ref, buf_ref,
             wf_ref, bf_ref, out_ref):
        wf = wf_ref[...]
        gu = ue_ref[...]
        gi = ie_ref[...]
        wf_u = wf[0:U_DIM, :]
        wf_t = wf[U_DIM:2 * U_DIM, :]
        wf_i = wf[2 * U_DIM:, :]
        du_lo = jnp.dot(gu[:, :U_DIM], wf_u, preferred_element_type=jnp.float32)
        du_hi = jnp.dot(gu[:, U_DIM:], wf_u, preferred_element_type=jnp.float32)
        di_lo = jnp.dot(gi[:, :I_DIM], wf_i, preferred_element_type=jnp.float32)
        di_hi = jnp.dot(gi[:, I_DIM:], wf_i, preferred_element_type=jnp.float32)
        hu = uh_ref[...].astype(jnp.float32)
        hi = ih_ref[...].astype(jnp.float32)
        t = jnp.dot(uf_ref[...], wuf_ref[...],
                    preferred_element_type=jnp.float32) + buf_ref[...]
        acc = du_lo + hu * (du_hi - du_lo)
        acc += di_lo + hi * (di_hi - di_lo)
        acc += jnp.dot(t, wf_t, preferred_element_type=jnp.float32)
        out_ref[...] = acc + bf_ref[...]

    return pl.pallas_call(
        body,
        grid=(B // blk,),
        in_specs=[
            pl.BlockSpec((blk, 2 * U_DIM), lambda i: (i, 0)),
            pl.BlockSpec((blk, 1), lambda i: (i, 0)),
            pl.BlockSpec((blk, ufd), lambda i: (i, 0)),
            pl.BlockSpec((blk, 2 * I_DIM), lambda i: (i, 0)),
            pl.BlockSpec((blk, 1), lambda i: (i, 0)),
            pl.BlockSpec((ufd, U_DIM), lambda i: (0, 0)),
            pl.BlockSpec((1, U_DIM), lambda i: (0, 0)),
            pl.BlockSpec((2 * U_DIM + I_DIM, n_tasks), lambda i: (0, 0)),
            pl.BlockSpec((1, n_tasks), lambda i: (0, 0)),
        ],
        out_specs=pl.BlockSpec((blk, n_tasks), lambda i: (i, 0)),
        out_shape=jax.ShapeDtypeStruct((B, n_tasks), jnp.float32),
    )(ue2, uhalf, uf, ie2, ihalf, W_uf, b_uf, W_final, b_final)


def kernel(user_id, user_features, item_id, user_table, item_table,
           W_uf, b_uf, W_final, b_final):
    uid = user_id.astype(jnp.int32)
    iid = item_id.astype(jnp.int32)
    ut2 = user_table.reshape(-1, 2 * U_DIM)
    it2 = item_table.reshape(-1, 2 * I_DIM)
    ue2, ie2 = _sc_gather(ut2, uid >> 1, it2, iid >> 1)
    uhalf = (uid & 1).reshape(-1, 1).astype(jnp.float32)
    ihalf = (iid & 1).reshape(-1, 1).astype(jnp.float32)
    return _tc_combine(ue2, uhalf, user_features, ie2, ihalf, W_uf,
                       b_uf.reshape(1, -1), W_final, b_final.reshape(1, -1))


# speedup vs baseline: 1.5266x; 1.5266x over previous
import functools

import jax
import jax.numpy as jnp
from jax import lax
from jax.experimental import pallas as pl
from jax.experimental.pallas import tpu as pltpu
from jax.experimental.pallas import tpu_sc as plsc

U_DIM = 64
NT = 3
VBLK = 2048
RBLK = VBLK // 128


def _tc_project(tblT, wT):
    D, V = tblT.shape
    nblk = pl.cdiv(V, VBLK)

    def body(t_ref, w_ref, o0_ref, o1_ref, o2_ref):
        p = lax.dot_general(w_ref[...], t_ref[...], (((1,), (0,)), ((), ())),
                            preferred_element_type=jnp.float32)
        o0_ref[...] = p[0:1, :].reshape(RBLK, 128)
        o1_ref[...] = p[1:2, :].reshape(RBLK, 128)
        o2_ref[...] = p[2:3, :].reshape(RBLK, 128)

    osd = jax.ShapeDtypeStruct((nblk * RBLK, 128), jnp.float32)
    ospec = pl.BlockSpec((RBLK, 128), lambda g: (g, 0))
    return pl.pallas_call(
        body,
        grid=(nblk,),
        in_specs=[
            pl.BlockSpec((D, VBLK), lambda g: (0, g)),
            pl.BlockSpec((NT, D), lambda g: (0, 0)),
        ],
        out_specs=(ospec, ospec, ospec),
        out_shape=(osd, osd, osd),
    )(tblT, wT)


def _sc_gather(pu, uidx, pi, iidx):
    B = uidx.shape[0]
    info = plsc.get_sparse_core_info()
    NC, NS = info.num_cores, info.num_subcores
    NW = NC * NS
    bpw = B // NW
    mesh = plsc.VectorSubcoreMesh(core_axis_name="c", subcore_axis_name="s")
    osd = jax.ShapeDtypeStruct((B, 128), jnp.float32)

    @functools.partial(
        pl.kernel,
        mesh=mesh,
        compiler_params=pltpu.CompilerParams(use_tc_tiling_on_sc=True),
        out_type=(osd,) * (2 * NT),
        scratch_types=[
            pltpu.VMEM((bpw,), jnp.int32),
            pltpu.VMEM((bpw,), jnp.int32),
            pltpu.VMEM((bpw, 128), jnp.float32),
            pltpu.SemaphoreType.DMA,
        ],
    )
    def gather_k(pu0, pu1, pu2, uid_hbm, pi0, pi1, pi2, iid_hbm,
                 gu0, gu1, gu2, gi0, gi1, gi2, uidx_v, iidx_v, rows_v, sem):
        wid = lax.axis_index("s") * NC + lax.axis_index("c")
        base = wid * bpw
        pltpu.sync_copy(uid_hbm.at[pl.ds(base, bpw)], uidx_v)
        pltpu.sync_copy(iid_hbm.at[pl.ds(base, bpw)], iidx_v)
        for src, dst, idx_v in ((pu0, gu0, uidx_v), (pu1, gu1, uidx_v),
                                (pu2, gu2, uidx_v), (pi0, gi0, iidx_v),
                                (pi1, gi1, iidx_v), (pi2, gi2, iidx_v)):
            pltpu.async_copy(src.at[idx_v], rows_v, sem).wait()
            pltpu.sync_copy(rows_v, dst.at[pl.ds(base, bpw)])

    return gather_k(pu[0], pu[1], pu[2], uidx, pi[0], pi[1], pi[2], iidx)


def _tc_combine(gu, ulane, gi, ilane, uf, W_uf, b_uf, W_final, b_final):
    B, ufd = uf.shape
    blk = 2048
    grid = (B // blk,)

    def body(gu0_ref, gu1_ref, gu2_ref, ul_ref, gi0_ref, gi1_ref, gi2_ref,
             il_ref, uf_ref, wuf_ref, buf_ref, wf_ref, bf_ref, out_ref):
        lane = lax.broadcasted_iota(jnp.int32, (blk, 128), 1)
        um = lane == ul_ref[...]
        im = lane == il_ref[...]

        def sel(m, g_ref):
            return jnp.sum(jnp.where(m, g_ref[...], 0.0), axis=1,
                           keepdims=True)

        emb = jnp.concatenate(
            [sel(um, gu0_ref), sel(um, gu1_ref), sel(um, gu2_ref)], axis=1)
        emb += jnp.concatenate(
            [sel(im, gi0_ref), sel(im, gi1_ref), sel(im, gi2_ref)], axis=1)
        t = jnp.dot(uf_ref[...], wuf_ref[...],
                    preferred_element_type=jnp.float32) + buf_ref[...]
        emb += jnp.dot(t, wf_ref[...][U_DIM:2 * U_DIM, :],
                       preferred_element_type=jnp.float32)
        out_ref[...] = emb + bf_ref[...]

    gspec = pl.BlockSpec((blk, 128), lambda i: (i, 0))
    lspec = pl.BlockSpec((blk, 1), lambda i: (i, 0))
    return pl.pallas_call(
        body,
        grid=grid,
        in_specs=[
            gspec, gspec, gspec, lspec, gspec, gspec, gspec, lspec,
            pl.BlockSpec((blk, ufd), lambda i: (i, 0)),
            pl.BlockSpec((ufd, U_DIM), lambda i: (0, 0)),
            pl.BlockSpec((1, U_DIM), lambda i: (0, 0)),
            pl.BlockSpec((3 * U_DIM, NT), lambda i: (0, 0)),
            pl.BlockSpec((1, NT), lambda i: (0, 0)),
        ],
        out_specs=pl.BlockSpec((blk, NT), lambda i: (i, 0)),
        out_shape=jax.ShapeDtypeStruct((B, NT), jnp.float32),
    )(gu[0], gu[1], gu[2], ulane, gi[0], gi[1], gi[2], ilane, uf,
      W_uf, b_uf, W_final, b_final)


def kernel(user_id, user_features, item_id, user_table, item_table,
           W_uf, b_uf, W_final, b_final):
    uid = user_id.astype(jnp.int32)
    iid = item_id.astype(jnp.int32)
    pu = _tc_project(user_table.T, W_final[0:U_DIM, :].T)
    pi = _tc_project(item_table.T, W_final[2 * U_DIM:, :].T)
    g = _sc_gather(pu, uid >> 7, pi, iid >> 7)
    ulane = (uid & 127).reshape(-1, 1)
    ilane = (iid & 127).reshape(-1, 1)
    return _tc_combine(g[:3], ulane, g[3:], ilane, user_features, W_uf,
                       b_uf.reshape(1, -1), W_final, b_final.reshape(1, -1))


# speedup vs baseline: 2.8012x; 1.8349x over previous
import functools

import jax
import jax.numpy as jnp
from jax import lax
from jax.experimental import pallas as pl
from jax.experimental.pallas import tpu as pltpu
from jax.experimental.pallas import tpu_sc as plsc

U_DIM = 64
NT = 3
VBLK = 8192
RBLK = VBLK // 128


def _tc_project(tblT, wT):
    D, V = tblT.shape
    nblk = pl.cdiv(V, VBLK)

    def body(t_ref, w_ref, o0_ref, o1_ref, o2_ref):
        p = lax.dot_general(w_ref[...], t_ref[...], (((1,), (0,)), ((), ())),
                            preferred_element_type=jnp.float32)
        o0_ref[...] = p[0:1, :].reshape(RBLK, 128)
        o1_ref[...] = p[1:2, :].reshape(RBLK, 128)
        o2_ref[...] = p[2:3, :].reshape(RBLK, 128)

    osd = jax.ShapeDtypeStruct((nblk * RBLK, 128), jnp.float32)
    ospec = pl.BlockSpec((RBLK, 128), lambda g: (g, 0))
    return pl.pallas_call(
        body,
        grid=(nblk,),
        in_specs=[
            pl.BlockSpec((D, VBLK), lambda g: (0, g)),
            pl.BlockSpec((NT, D), lambda g: (0, 0)),
        ],
        out_specs=(ospec, ospec, ospec),
        out_shape=(osd, osd, osd),
    )(tblT, wT)


def _sc_gather(pu, uidx, pi, iidx):
    B = uidx.shape[0]
    info = plsc.get_sparse_core_info()
    NC, NS = info.num_cores, info.num_subcores
    NW = NC * NS
    bpw = B // NW
    mesh = plsc.VectorSubcoreMesh(core_axis_name="c", subcore_axis_name="s")
    osd = jax.ShapeDtypeStruct((B, 128), jnp.float32)

    @functools.partial(
        pl.kernel,
        mesh=mesh,
        compiler_params=pltpu.CompilerParams(use_tc_tiling_on_sc=True),
        out_type=(osd,) * (2 * NT),
        scratch_types=[
            pltpu.VMEM((bpw,), jnp.int32),
            pltpu.VMEM((bpw,), jnp.int32),
            pltpu.VMEM((bpw, 128), jnp.float32),
            pltpu.SemaphoreType.DMA,
        ],
    )
    def gather_k(pu0, pu1, pu2, uid_hbm, pi0, pi1, pi2, iid_hbm,
                 gu0, gu1, gu2, gi0, gi1, gi2, uidx_v, iidx_v, rows_v, sem):
        wid = lax.axis_index("s") * NC + lax.axis_index("c")
        base = wid * bpw
        pltpu.sync_copy(uid_hbm.at[pl.ds(base, bpw)], uidx_v)
        pltpu.sync_copy(iid_hbm.at[pl.ds(base, bpw)], iidx_v)
        for src, dst, idx_v in ((pu0, gu0, uidx_v), (pu1, gu1, uidx_v),
                                (pu2, gu2, uidx_v), (pi0, gi0, iidx_v),
                                (pi1, gi1, iidx_v), (pi2, gi2, iidx_v)):
            pltpu.async_copy(src.at[idx_v], rows_v, sem).wait()
            pltpu.sync_copy(rows_v, dst.at[pl.ds(base, bpw)])

    return gather_k(pu[0], pu[1], pu[2], uidx, pi[0], pi[1], pi[2], iidx)


def _tc_combine(gu, ulane, gi, ilane, uf, W_uf, b_uf, W_final, b_final):
    B, ufd = uf.shape
    blk = 2048
    grid = (B // blk,)

    def body(gu0_ref, gu1_ref, gu2_ref, ul_ref, gi0_ref, gi1_ref, gi2_ref,
             il_ref, uf_ref, wuf_ref, buf_ref, wf_ref, bf_ref, out_ref):
        lane = lax.broadcasted_iota(jnp.int32, (blk, 128), 1)
        um = lane == ul_ref[...]
        im = lane == il_ref[...]

        def sel(m, g_ref):
            return jnp.sum(jnp.where(m, g_ref[...], 0.0), axis=1,
                           keepdims=True)

        emb = jnp.concatenate(
            [sel(um, gu0_ref), sel(um, gu1_ref), sel(um, gu2_ref)], axis=1)
        emb += jnp.concatenate(
            [sel(im, gi0_ref), sel(im, gi1_ref), sel(im, gi2_ref)], axis=1)
        t = jnp.dot(uf_ref[...], wuf_ref[...],
                    preferred_element_type=jnp.float32) + buf_ref[...]
        emb += jnp.dot(t, wf_ref[...][U_DIM:2 * U_DIM, :],
                       preferred_element_type=jnp.float32)
        out_ref[...] = emb + bf_ref[...]

    gspec = pl.BlockSpec((blk, 128), lambda i: (i, 0))
    lspec = pl.BlockSpec((blk, 1), lambda i: (i, 0))
    return pl.pallas_call(
        body,
        grid=grid,
        in_specs=[
            gspec, gspec, gspec, lspec, gspec, gspec, gspec, lspec,
            pl.BlockSpec((blk, ufd), lambda i: (i, 0)),
            pl.BlockSpec((ufd, U_DIM), lambda i: (0, 0)),
            pl.BlockSpec((1, U_DIM), lambda i: (0, 0)),
            pl.BlockSpec((3 * U_DIM, NT), lambda i: (0, 0)),
            pl.BlockSpec((1, NT), lambda i: (0, 0)),
        ],
        out_specs=pl.BlockSpec((blk, NT), lambda i: (i, 0)),
        out_shape=jax.ShapeDtypeStruct((B, NT), jnp.float32),
    )(gu[0], gu[1], gu[2], ulane, gi[0], gi[1], gi[2], ilane, uf,
      W_uf, b_uf, W_final, b_final)


def kernel(user_id, user_features, item_id, user_table, item_table,
           W_uf, b_uf, W_final, b_final):
    uid = user_id.astype(jnp.int32)
    iid = item_id.astype(jnp.int32)
    pu = _tc_project(user_table.T, W_final[0:U_DIM, :].T)
    pi = _tc_project(item_table.T, W_final[2 * U_DIM:, :].T)
    g = _sc_gather(pu, uid >> 7, pi, iid >> 7)
    ulane = (uid & 127).reshape(-1, 1)
    ilane = (iid & 127).reshape(-1, 1)
    return _tc_combine(g[:3], ulane, g[3:], ilane, user_features, W_uf,
                       b_uf.reshape(1, -1), W_final, b_final.reshape(1, -1))


# speedup vs baseline: 3.4747x; 1.2404x over previous
import functools

import jax
import jax.numpy as jnp
from jax import lax
from jax.experimental import pallas as pl
from jax.experimental.pallas import tpu as pltpu
from jax.experimental.pallas import tpu_sc as plsc

U_DIM = 64
NT = 3
VBLK = 32768
RBLK = VBLK // 128


def _tc_project(tblT, wT):
    D, V = tblT.shape
    nblk = pl.cdiv(V, VBLK)

    def body(t_ref, w_ref, o0_ref, o1_ref, o2_ref):
        p = lax.dot_general(w_ref[...], t_ref[...], (((1,), (0,)), ((), ())),
                            preferred_element_type=jnp.float32)
        o0_ref[...] = p[0:1, :].reshape(RBLK, 128)
        o1_ref[...] = p[1:2, :].reshape(RBLK, 128)
        o2_ref[...] = p[2:3, :].reshape(RBLK, 128)

    osd = jax.ShapeDtypeStruct((nblk * RBLK, 128), jnp.float32)
    ospec = pl.BlockSpec((RBLK, 128), lambda g: (g, 0))
    return pl.pallas_call(
        body,
        grid=(nblk,),
        in_specs=[
            pl.BlockSpec((D, VBLK), lambda g: (0, g)),
            pl.BlockSpec((NT, D), lambda g: (0, 0)),
        ],
        out_specs=(ospec, ospec, ospec),
        out_shape=(osd, osd, osd),
    )(tblT, wT)


def _sc_gather(pu, uidx, pi, iidx):
    B = uidx.shape[0]
    info = plsc.get_sparse_core_info()
    NC, NS = info.num_cores, info.num_subcores
    NW = NC * NS
    bpw = B // NW
    mesh = plsc.VectorSubcoreMesh(core_axis_name="c", subcore_axis_name="s")
    osd = jax.ShapeDtypeStruct((B, 128), jnp.float32)

    @functools.partial(
        pl.kernel,
        mesh=mesh,
        compiler_params=pltpu.CompilerParams(use_tc_tiling_on_sc=True),
        out_type=(osd,) * (2 * NT),
        scratch_types=[
            pltpu.VMEM((bpw,), jnp.int32),
            pltpu.VMEM((bpw,), jnp.int32),
            pltpu.VMEM((bpw, 128), jnp.float32),
            pltpu.SemaphoreType.DMA,
        ],
    )
    def gather_k(pu0, pu1, pu2, uid_hbm, pi0, pi1, pi2, iid_hbm,
                 gu0, gu1, gu2, gi0, gi1, gi2, uidx_v, iidx_v, rows_v, sem):
        wid = lax.axis_index("s") * NC + lax.axis_index("c")
        base = wid * bpw
        pltpu.sync_copy(uid_hbm.at[pl.ds(base, bpw)], uidx_v)
        pltpu.sync_copy(iid_hbm.at[pl.ds(base, bpw)], iidx_v)
        for src, dst, idx_v in ((pu0, gu0, uidx_v), (pu1, gu1, uidx_v),
                                (pu2, gu2, uidx_v), (pi0, gi0, iidx_v),
                                (pi1, gi1, iidx_v), (pi2, gi2, iidx_v)):
            pltpu.async_copy(src.at[idx_v], rows_v, sem).wait()
            pltpu.sync_copy(rows_v, dst.at[pl.ds(base, bpw)])

    return gather_k(pu[0], pu[1], pu[2], uidx, pi[0], pi[1], pi[2], iidx)


def _tc_combine(gu, ulane, gi, ilane, uf, W_uf, b_uf, W_final, b_final):
    B, ufd = uf.shape
    blk = 2048
    grid = (B // blk,)

    def body(gu0_ref, gu1_ref, gu2_ref, ul_ref, gi0_ref, gi1_ref, gi2_ref,
             il_ref, uf_ref, wuf_ref, buf_ref, wf_ref, bf_ref, out_ref):
        lane = lax.broadcasted_iota(jnp.int32, (blk, 128), 1)
        um = lane == ul_ref[...]
        im = lane == il_ref[...]

        def sel(m, g_ref):
            return jnp.sum(jnp.where(m, g_ref[...], 0.0), axis=1,
                           keepdims=True)

        emb = jnp.concatenate(
            [sel(um, gu0_ref), sel(um, gu1_ref), sel(um, gu2_ref)], axis=1)
        emb += jnp.concatenate(
            [sel(im, gi0_ref), sel(im, gi1_ref), sel(im, gi2_ref)], axis=1)
        t = jnp.dot(uf_ref[...], wuf_ref[...],
                    preferred_element_type=jnp.float32) + buf_ref[...]
        emb += jnp.dot(t, wf_ref[...][U_DIM:2 * U_DIM, :],
                       preferred_element_type=jnp.float32)
        out_ref[...] = emb + bf_ref[...]

    gspec = pl.BlockSpec((blk, 128), lambda i: (i, 0))
    lspec = pl.BlockSpec((blk, 1), lambda i: (i, 0))
    return pl.pallas_call(
        body,
        grid=grid,
        in_specs=[
            gspec, gspec, gspec, lspec, gspec, gspec, gspec, lspec,
            pl.BlockSpec((blk, ufd), lambda i: (i, 0)),
            pl.BlockSpec((ufd, U_DIM), lambda i: (0, 0)),
            pl.BlockSpec((1, U_DIM), lambda i: (0, 0)),
            pl.BlockSpec((3 * U_DIM, NT), lambda i: (0, 0)),
            pl.BlockSpec((1, NT), lambda i: (0, 0)),
        ],
        out_specs=pl.BlockSpec((blk, NT), lambda i: (i, 0)),
        out_shape=jax.ShapeDtypeStruct((B, NT), jnp.float32),
    )(gu[0], gu[1], gu[2], ulane, gi[0], gi[1], gi[2], ilane, uf,
      W_uf, b_uf, W_final, b_final)


def kernel(user_id, user_features, item_id, user_table, item_table,
           W_uf, b_uf, W_final, b_final):
    uid = user_id.astype(jnp.int32)
    iid = item_id.astype(jnp.int32)
    pu = _tc_project(user_table.T, W_final[0:U_DIM, :].T)
    pi = _tc_project(item_table.T, W_final[2 * U_DIM:, :].T)
    g = _sc_gather(pu, uid >> 7, pi, iid >> 7)
    ulane = (uid & 127).reshape(-1, 1)
    ilane = (iid & 127).reshape(-1, 1)
    return _tc_combine(g[:3], ulane, g[3:], ilane, user_features, W_uf,
                       b_uf.reshape(1, -1), W_final, b_final.reshape(1, -1))


# speedup vs baseline: 3.5037x; 1.0083x over previous
import functools

import jax
import jax.numpy as jnp
from jax import lax
from jax.experimental import pallas as pl
from jax.experimental.pallas import tpu as pltpu
from jax.experimental.pallas import tpu_sc as plsc

U_DIM = 64
NT = 3
VBLK = 32768
RBLK = VBLK // 128


def _tc_project(tblT, wT):
    D, V = tblT.shape
    nblk = pl.cdiv(V, VBLK)

    def body(t_ref, w_ref, o0_ref, o1_ref, o2_ref):
        p = lax.dot_general(w_ref[...], t_ref[...], (((1,), (0,)), ((), ())),
                            preferred_element_type=jnp.float32)
        o0_ref[...] = p[0:1, :].reshape(RBLK, 128)
        o1_ref[...] = p[1:2, :].reshape(RBLK, 128)
        o2_ref[...] = p[2:3, :].reshape(RBLK, 128)

    osd = jax.ShapeDtypeStruct((nblk * RBLK, 128), jnp.float32)
    ospec = pl.BlockSpec((RBLK, 128), lambda g: (g, 0))
    return pl.pallas_call(
        body,
        grid=(nblk,),
        in_specs=[
            pl.BlockSpec((D, VBLK), lambda g: (0, g)),
            pl.BlockSpec((NT, D), lambda g: (0, 0)),
        ],
        out_specs=(ospec, ospec, ospec),
        out_shape=(osd, osd, osd),
    )(tblT, wT)


def _sc_gather(pu, uid, pi, iid):
    B = uid.shape[0]
    info = plsc.get_sparse_core_info()
    NC, NS = info.num_cores, info.num_subcores
    NW = NC * NS
    bpw = B // NW
    nchunk = bpw // 16
    mesh = plsc.VectorSubcoreMesh(core_axis_name="c", subcore_axis_name="s")
    osd = jax.ShapeDtypeStruct((B,), jnp.float32)

    @functools.partial(
        pl.kernel,
        mesh=mesh,
        compiler_params=pltpu.CompilerParams(use_tc_tiling_on_sc=True,
                                             needs_layout_passes=False),
        out_type=(osd,) * (2 * NT),
        scratch_types=[
            pltpu.VMEM((bpw,), jnp.int32),
            pltpu.VMEM((bpw,), jnp.int32),
            pltpu.VMEM((bpw,), jnp.int32),
            pltpu.VMEM((bpw,), jnp.float32),
            pltpu.VMEM((bpw, 128), jnp.float32),
            pltpu.SemaphoreType.DMA,
        ],
    )
    def gather_k(pu0, pu1, pu2, uid_hbm, pi0, pi1, pi2, iid_hbm,
                 eu0, eu1, eu2, ei0, ei1, ei2,
                 ids_v, row_v, lane_v, val_v, rows_v, sem):
        wid = lax.axis_index("s") * NC + lax.axis_index("c")
        base = wid * bpw
        for id_hbm, srcs, dsts in (
            (uid_hbm, (pu0, pu1, pu2), (eu0, eu1, eu2)),
            (iid_hbm, (pi0, pi1, pi2), (ei0, ei1, ei2)),
        ):
            pltpu.sync_copy(id_hbm.at[pl.ds(base, bpw)], ids_v)
            for j in range(nchunk):
                ids16 = ids_v[pl.ds(16 * j, 16)]
                row_v[pl.ds(16 * j, 16)] = ids16 >> 7
                lane_v[pl.ds(16 * j, 16)] = ids16 & 127
            for src, dst in zip(srcs, dsts):
                pltpu.async_copy(src.at[row_v], rows_v, sem).wait()
                for j in range(nchunk):
                    r16 = lax.iota(jnp.int32, 16) + 16 * j
                    vals = plsc.load_gather(
                        rows_v, [r16, lane_v[pl.ds(16 * j, 16)]])
                    val_v[pl.ds(16 * j, 16)] = vals
                pltpu.sync_copy(val_v, dst.at[pl.ds(base, bpw)])

    return gather_k(pu[0], pu[1], pu[2], uid, pi[0], pi[1], pi[2], iid)


def _tc_combine(eu, ei, uf, W_uf, b_uf, W_final, b_final):
    B, ufd = uf.shape
    blk = 2048
    espec = pl.BlockSpec((blk, 1), lambda i: (i, 0))

    def body(eu0_ref, eu1_ref, eu2_ref, ei0_ref, ei1_ref, ei2_ref,
             uf_ref, wuf_ref, buf_ref, wf_ref, bf_ref, out_ref):
        emb = jnp.concatenate(
            [eu0_ref[...] + ei0_ref[...],
             eu1_ref[...] + ei1_ref[...],
             eu2_ref[...] + ei2_ref[...]], axis=1)
        t = jnp.dot(uf_ref[...], wuf_ref[...],
                    preferred_element_type=jnp.float32) + buf_ref[...]
        emb += jnp.dot(t, wf_ref[...][U_DIM:2 * U_DIM, :],
                       preferred_element_type=jnp.float32)
        out_ref[...] = emb + bf_ref[...]

    return pl.pallas_call(
        body,
        grid=(B // blk,),
        in_specs=[
            espec, espec, espec, espec, espec, espec,
            pl.BlockSpec((blk, ufd), lambda i: (i, 0)),
            pl.BlockSpec((ufd, U_DIM), lambda i: (0, 0)),
            pl.BlockSpec((1, U_DIM), lambda i: (0, 0)),
            pl.BlockSpec((3 * U_DIM, NT), lambda i: (0, 0)),
            pl.BlockSpec((1, NT), lambda i: (0, 0)),
        ],
        out_specs=pl.BlockSpec((blk, NT), lambda i: (i, 0)),
        out_shape=jax.ShapeDtypeStruct((B, NT), jnp.float32),
    )(eu[0].reshape(-1, 1), eu[1].reshape(-1, 1), eu[2].reshape(-1, 1),
      ei[0].reshape(-1, 1), ei[1].reshape(-1, 1), ei[2].reshape(-1, 1),
      uf, W_uf, b_uf, W_final, b_final)


def kernel(user_id, user_features, item_id, user_table, item_table,
           W_uf, b_uf, W_final, b_final):
    uid = user_id.astype(jnp.int32)
    iid = item_id.astype(jnp.int32)
    pu = _tc_project(user_table.T, W_final[0:U_DIM, :].T)
    pi = _tc_project(item_table.T, W_final[2 * U_DIM:, :].T)
    g = _sc_gather(pu, uid, pi, iid)
    return _tc_combine(g[:3], g[3:], user_features, W_uf,
                       b_uf.reshape(1, -1), W_final, b_final.reshape(1, -1))


# speedup vs baseline: 4.6574x; 1.3293x over previous
import functools

import jax
import jax.numpy as jnp
from jax import lax
from jax.experimental import pallas as pl
from jax.experimental.pallas import tpu as pltpu
from jax.experimental.pallas import tpu_sc as plsc

U_DIM = 64
NT = 3
VBLK = 32768
RBLK = VBLK // 128


def _tc_project(tblT, wT):
    D, V = tblT.shape
    nblk = pl.cdiv(V, VBLK)

    def body(t_ref, w_ref, o0_ref, o1_ref, o2_ref):
        p = lax.dot_general(w_ref[...], t_ref[...], (((1,), (0,)), ((), ())),
                            preferred_element_type=jnp.float32)
        o0_ref[...] = p[0:1, :].reshape(RBLK, 128)
        o1_ref[...] = p[1:2, :].reshape(RBLK, 128)
        o2_ref[...] = p[2:3, :].reshape(RBLK, 128)

    osd = jax.ShapeDtypeStruct((nblk * RBLK, 128), jnp.float32)
    ospec = pl.BlockSpec((RBLK, 128), lambda g: (g, 0))
    return pl.pallas_call(
        body,
        grid=(nblk,),
        in_specs=[
            pl.BlockSpec((D, VBLK), lambda g: (0, g)),
            pl.BlockSpec((NT, D), lambda g: (0, 0)),
        ],
        out_specs=(ospec, ospec, ospec),
        out_shape=(osd, osd, osd),
    )(tblT, wT)


def _sc_gather(pu, uid, pi, iid):
    B = uid.shape[0]
    info = plsc.get_sparse_core_info()
    NC, NS = info.num_cores, info.num_subcores
    NW = NC * NS
    bpw = B // NW
    nchunk = bpw // 16
    mesh = plsc.VectorSubcoreMesh(core_axis_name="c", subcore_axis_name="s")
    osd = jax.ShapeDtypeStruct((B,), jnp.float32)

    @functools.partial(
        pl.kernel,
        mesh=mesh,
        compiler_params=pltpu.CompilerParams(use_tc_tiling_on_sc=True,
                                             needs_layout_passes=False),
        out_type=(osd,) * (2 * NT),
        scratch_types=[
            pltpu.VMEM((bpw,), jnp.int32),
            pltpu.VMEM((bpw,), jnp.int32),
            pltpu.VMEM((bpw,), jnp.int32),
            pltpu.VMEM((bpw,), jnp.float32),
            pltpu.VMEM((bpw, 128), jnp.float32),
            pltpu.SemaphoreType.DMA,
        ],
    )
    def gather_k(pu0, pu1, pu2, uid_hbm, pi0, pi1, pi2, iid_hbm,
                 eu0, eu1, eu2, ei0, ei1, ei2,
                 ids_v, row_v, lane_v, val_v, rows_v, sem):
        wid = lax.axis_index("s") * NC + lax.axis_index("c")
        base = wid * bpw
        for id_hbm, srcs, dsts in (
            (uid_hbm, (pu0, pu1, pu2), (eu0, eu1, eu2)),
            (iid_hbm, (pi0, pi1, pi2), (ei0, ei1, ei2)),
        ):
            pltpu.sync_copy(id_hbm.at[pl.ds(base, bpw)], ids_v)
            for j in range(nchunk):
                ids16 = ids_v[pl.ds(16 * j, 16)]
                row_v[pl.ds(16 * j, 16)] = ids16 >> 7
                lane_v[pl.ds(16 * j, 16)] = ids16 & 127
            for src, dst in zip(srcs, dsts):
                pltpu.async_copy(src.at[row_v], rows_v, sem).wait()
                for j in range(nchunk):
                    r16 = lax.iota(jnp.int32, 16) + 16 * j
                    vals = plsc.load_gather(
                        rows_v, [r16, lane_v[pl.ds(16 * j, 16)]])
                    val_v[pl.ds(16 * j, 16)] = vals
                pltpu.sync_copy(val_v, dst.at[pl.ds(base, bpw)])

    return gather_k(pu[0], pu[1], pu[2], uid, pi[0], pi[1], pi[2], iid)


def _tc_combine(eu, ei, uf, W_ufT, b_uf, W_finalT, b_finalT):
    B, ufd = uf.shape
    blk = 2048
    espec = pl.BlockSpec((blk // 128, 128), lambda i: (i, 0))

    def body(eu0_ref, eu1_ref, eu2_ref, ei0_ref, ei1_ref, ei2_ref,
             uf_ref, wufT_ref, buf_ref, wfT_ref, bfT_ref, out_ref):
        rows = [
            (u_ref[...] + i_ref[...]).reshape(1, blk)
            for u_ref, i_ref in ((eu0_ref, ei0_ref), (eu1_ref, ei1_ref),
                                 (eu2_ref, ei2_ref))
        ]
        emb = jnp.concatenate(rows, axis=0)
        tT = lax.dot_general(wufT_ref[...], uf_ref[...],
                             (((1,), (1,)), ((), ())),
                             preferred_element_type=jnp.float32) + buf_ref[...]
        emb += jnp.dot(wfT_ref[...][:, U_DIM:2 * U_DIM], tT,
                       preferred_element_type=jnp.float32)
        out_ref[...] = emb + bfT_ref[...]

    outT = pl.pallas_call(
        body,
        grid=(B // blk,),
        in_specs=[
            espec, espec, espec, espec, espec, espec,
            pl.BlockSpec((blk, ufd), lambda i: (i, 0)),
            pl.BlockSpec((U_DIM, ufd), lambda i: (0, 0)),
            pl.BlockSpec((U_DIM, 1), lambda i: (0, 0)),
            pl.BlockSpec((NT, 3 * U_DIM), lambda i: (0, 0)),
            pl.BlockSpec((NT, 1), lambda i: (0, 0)),
        ],
        out_specs=pl.BlockSpec((NT, blk), lambda i: (0, i)),
        out_shape=jax.ShapeDtypeStruct((NT, B), jnp.float32),
    )(eu[0].reshape(128, -1), eu[1].reshape(128, -1), eu[2].reshape(128, -1),
      ei[0].reshape(128, -1), ei[1].reshape(128, -1), ei[2].reshape(128, -1),
      uf, W_ufT, b_uf, W_finalT, b_finalT)
    return outT.T


def kernel(user_id, user_features, item_id, user_table, item_table,
           W_uf, b_uf, W_final, b_final):
    uid = user_id.astype(jnp.int32)
    iid = item_id.astype(jnp.int32)
    pu = _tc_project(user_table.T, W_final[0:U_DIM, :].T)
    pi = _tc_project(item_table.T, W_final[2 * U_DIM:, :].T)
    g = _sc_gather(pu, uid, pi, iid)
    return _tc_combine(g[:3], g[3:], user_features, W_uf.T,
                       b_uf.reshape(-1, 1), W_final.T,
                       b_final.reshape(-1, 1))
